# baseline (speedup 1.0000x reference)
import jax
import jax.numpy as jnp
from jax import lax
from jax.experimental import pallas as pl
from jax.experimental.pallas import tpu as pltpu
from jax.experimental.pallas import tpu_sc as plsc

N = 10000
NP = 10240
D = 128
E = 320000

NC = 2
NS = 16
NW = NC * NS
CHUNK = 128
WROWS = 80
EROWS = NW * WROWS
EPAD = EROWS * CHUNK
HROWS = WROWS // 2
ROWS_PER_SUB = NP // NS
ZROWS = 32
SWIN = 8
DEGW = 128
DINVW = 16

_sc_mesh = plsc.VectorSubcoreMesh(
    core_axis_name="c", subcore_axis_name="s", num_cores=NC, num_subcores=NS
)


def _zero_vmem(buf, rows, width):
    zero = jnp.zeros((16,), jnp.float32)
    for r in range(rows):
        for j in range(width // 16):
            buf[r, pl.ds(j * 16, 16)] = zero


def _make_sc_degree(interpret=False):
    return pl.kernel(
        _sc_degree_body,
        out_type=jax.ShapeDtypeStruct((NC, NP, DEGW), jnp.float32),
        mesh=_sc_mesh,
        scratch_types=[
            pltpu.VMEM((WROWS, CHUNK), jnp.int32),
            pltpu.VMEM((CHUNK, DEGW), jnp.float32),
            pltpu.VMEM((ZROWS, DEGW), jnp.float32),
            pltpu.VMEM_SHARED((NP, DEGW), jnp.float32),
            pltpu.SemaphoreType.DMA,
        ],
        interpret=interpret,
    )


def _sc_degree_body(dst2_hbm, out_hbm, dst2d, ones_v, zbuf, acc, ssem):
    c = lax.axis_index("c")
    s = lax.axis_index("s")
    w = s * NC + c

    _zero_vmem(zbuf, ZROWS, DEGW)
    one = jnp.full((16,), 1.0, jnp.float32)
    for r in range(CHUNK):
        for j in range(DEGW // 16):
            ones_v[r, pl.ds(j * 16, 16)] = one

    def zloop(i, _):
        pltpu.sync_copy(zbuf, acc.at[pl.ds(s * ROWS_PER_SUB + i * ZROWS, ZROWS)])
        return ()

    lax.fori_loop(0, ROWS_PER_SUB // ZROWS, zloop, ())
    pltpu.sync_copy(dst2_hbm.at[pl.ds(w * WROWS, WROWS)], dst2d)
    plsc.subcore_barrier()

    def fire(i, _):
        pltpu.async_copy(ones_v, acc.at[dst2d.at[i]], ssem, add=True)

        @pl.when(i >= SWIN)
        def _drain_old():
            pltpu.make_async_copy(ones_v, acc.at[dst2d.at[i - SWIN]], ssem).wait()

        return ()

    lax.fori_loop(0, WROWS, fire, ())

    def drain(i, _):
        pltpu.make_async_copy(ones_v, acc.at[dst2d.at[i]], ssem).wait()
        return ()

    lax.fori_loop(WROWS - SWIN, WROWS, drain, ())

    plsc.subcore_barrier()
    pltpu.sync_copy(
        acc.at[pl.ds(s * ROWS_PER_SUB, ROWS_PER_SUB)],
        out_hbm.at[c, pl.ds(s * ROWS_PER_SUB, ROWS_PER_SUB)],
    )


def _make_sc_aggregate(interpret=False):
    return pl.kernel(
        _sc_aggregate_body,
        out_type=jax.ShapeDtypeStruct((NC, NP, D), jnp.float32),
        mesh=_sc_mesh,
        scratch_types=[
            pltpu.VMEM((HROWS, CHUNK), jnp.int32),
            pltpu.VMEM((HROWS, CHUNK), jnp.int32),
            pltpu.VMEM((CHUNK, D), jnp.float32),
            pltpu.VMEM((CHUNK, D), jnp.float32),
            pltpu.VMEM_SHARED((NP, D), jnp.float32),
            pltpu.SemaphoreType.DMA,
            pltpu.SemaphoreType.DMA,
        ],
        interpret=interpret,
    )


def _sc_aggregate_body(
    hs_hbm, src2_hbm, dst2_hbm, out_hbm,
    src2d, dst2d, rows0, rows1, acc, gsem, ssem,
):
    c = lax.axis_index("c")
    s = lax.axis_index("s")
    w = s * NC + c

    _zero_vmem(rows0, CHUNK, D)

    def zloop(i, _):
        pltpu.sync_copy(rows0, acc.at[pl.ds(s * ROWS_PER_SUB + i * CHUNK, CHUNK)])
        return ()

    lax.fori_loop(0, ROWS_PER_SUB // CHUNK, zloop, ())

    def gather(ci, buf):
        pltpu.async_copy(hs_hbm.at[src2d.at[ci]], buf, gsem)

    def wait_gather(ci, buf):
        pltpu.make_async_copy(hs_hbm.at[src2d.at[ci]], buf, gsem).wait()

    def scatter(ci, buf):
        pltpu.async_copy(buf, acc.at[dst2d.at[ci]], ssem, add=True)

    def wait_scatter(ci, buf):
        pltpu.make_async_copy(buf, acc.at[dst2d.at[ci]], ssem).wait()

    def pair(c2, _):
        c0 = 2 * c2
        c1 = c0 + 1
        wait_gather(c0, rows0)

        @pl.when(c2 > 0)
        def _wait_prev():
            wait_scatter(c0 - 1, rows1)

        gather(c1, rows1)
        scatter(c0, rows0)
        wait_gather(c1, rows1)
        wait_scatter(c0, rows0)

        @pl.when(c2 < HROWS // 2 - 1)
        def _prefetch_next():
            gather(c1 + 1, rows0)

        scatter(c1, rows1)
        return ()

    for h in range(WROWS // HROWS):
        pltpu.sync_copy(src2_hbm.at[pl.ds(w * WROWS + h * HROWS, HROWS)], src2d)
        pltpu.sync_copy(dst2_hbm.at[pl.ds(w * WROWS + h * HROWS, HROWS)], dst2d)
        if h == 0:
            plsc.subcore_barrier()
        gather(0, rows0)
        lax.fori_loop(0, HROWS // 2, pair, ())
        wait_scatter(HROWS - 1, rows1)

    plsc.subcore_barrier()
    pltpu.sync_copy(
        acc.at[pl.ds(s * ROWS_PER_SUB, ROWS_PER_SUB)],
        out_hbm.at[c, pl.ds(s * ROWS_PER_SUB, ROWS_PER_SUB)],
    )


_sc_degree = _make_sc_degree()
_sc_aggregate = _make_sc_aggregate()


ROWBLK = 1024
GRID = NP // ROWBLK

_row_spec = pl.BlockSpec((ROWBLK, D), lambda i: (i, 0))
_p_spec = pl.BlockSpec((ROWBLK, DEGW), lambda i: (i, 0))
_dinv_spec = pl.BlockSpec((ROWBLK, DINVW), lambda i: (i, 0))
_w_spec = pl.BlockSpec((D, D), lambda i: (0, 0))
_b_spec = pl.BlockSpec((1, D), lambda i: (0, 0))


def _leaky(y):
    return jnp.where(y >= 0, y, 0.01 * y)


def _tc_first_body(x_ref, w_ref, p0_ref, p1_ref, hs_ref, dinv_ref):
    deg = p0_ref[:, 0:1] + p1_ref[:, 0:1] + 1.0
    dinv = lax.rsqrt(jnp.maximum(deg, 1.0))
    hs_ref[...] = dinv * jnp.dot(
        x_ref[...], w_ref[...], preferred_element_type=jnp.float32
    )
    dinv_ref[...] = jnp.broadcast_to(dinv, (ROWBLK, DINVW))


def _tc_first(x, w1, p0, p1):
    return pl.pallas_call(
        _tc_first_body,
        grid=(GRID,),
        in_specs=[_row_spec, _w_spec, _p_spec, _p_spec],
        out_specs=[_row_spec, _dinv_spec],
        out_shape=[
            jax.ShapeDtypeStruct((NP, D), jnp.float32),
            jax.ShapeDtypeStruct((NP, DINVW), jnp.float32),
        ],
    )(x, w1, p0, p1)


def _tc_mid_body(p0_ref, p1_ref, hs_ref, b_ref, dinv_ref, w_ref, out_ref):
    dinv = dinv_ref[:, 0:1]
    y = dinv * (p0_ref[...] + p1_ref[...] + hs_ref[...]) + b_ref[...]
    xn = _leaky(y)
    out_ref[...] = dinv * jnp.dot(
        xn, w_ref[...], preferred_element_type=jnp.float32
    )


def _tc_mid(p0, p1, hs, b, dinvb, w):
    return pl.pallas_call(
        _tc_mid_body,
        grid=(GRID,),
        in_specs=[_row_spec, _row_spec, _row_spec, _b_spec, _dinv_spec, _w_spec],
        out_specs=_row_spec,
        out_shape=jax.ShapeDtypeStruct((NP, D), jnp.float32),
    )(p0, p1, hs, b, dinvb, w)


def _tc_last_body(p0_ref, p1_ref, hs_ref, b_ref, dinv_ref, out_ref):
    dinv = dinv_ref[:, 0:1]
    y = dinv * (p0_ref[...] + p1_ref[...] + hs_ref[...]) + b_ref[...]
    out_ref[...] = _leaky(y)


def _tc_last(p0, p1, hs, b, dinvb):
    return pl.pallas_call(
        _tc_last_body,
        grid=(GRID,),
        in_specs=[_row_spec, _row_spec, _row_spec, _b_spec, _dinv_spec],
        out_specs=_row_spec,
        out_shape=jax.ShapeDtypeStruct((NP, D), jnp.float32),
    )(p0, p1, hs, b, dinvb)


def kernel(x, edge_index, W1, b1, W2, b2, W3, b3, W4, b4):
    src = edge_index[0].astype(jnp.int32)
    dst = edge_index[1].astype(jnp.int32)
    pad = jnp.full((EPAD - E,), N, jnp.int32)
    src2 = jnp.concatenate([src, pad]).reshape(EROWS, CHUNK)
    dst2 = jnp.concatenate([dst, pad]).reshape(EROWS, CHUNK)
    xp = jnp.pad(x, ((0, NP - N), (0, 0)))

    dpart = _sc_degree(dst2)
    hs, dinvb = _tc_first(xp, W1, dpart[0], dpart[1])

    for w, b in ((W2, b1), (W3, b2), (W4, b3)):
        p = _sc_aggregate(hs, src2, dst2)
        hs = _tc_mid(p[0], p[1], hs, b.reshape(1, D), dinvb, w)

    p = _sc_aggregate(hs, src2, dst2)
    out = _tc_last(p[0], p[1], hs, b4.reshape(1, D), dinvb)
    return out[:N]


# speedup vs baseline: 2.7278x; 2.7278x over previous
import jax
import jax.numpy as jnp
from jax import lax
from jax.experimental import pallas as pl
from jax.experimental.pallas import tpu as pltpu
from jax.experimental.pallas import tpu_sc as plsc

N = 10000
NP = 10240
D = 128
E = 320000

NC = 2
NS = 16
NW = NC * NS
CHUNK = 128
WROWS = 80
EROWS = NW * WROWS
EPAD = EROWS * CHUNK
HROWS = WROWS // 2
ROWS_PER_SUB = NP // NS
ZROWS = 32
SWIN = 8
DEGW = 128
DINVW = 16

_sc_mesh = plsc.VectorSubcoreMesh(
    core_axis_name="c", subcore_axis_name="s", num_cores=NC, num_subcores=NS
)


def _zero_vmem(buf, rows, width):
    zero = jnp.zeros((16,), jnp.float32)
    for r in range(rows):
        for j in range(width // 16):
            buf[r, pl.ds(j * 16, 16)] = zero


def _make_sc_degree(interpret=False):
    return pl.kernel(
        _sc_degree_body,
        out_type=jax.ShapeDtypeStruct((NC, NP, DEGW), jnp.float32),
        mesh=_sc_mesh,
        scratch_types=[
            pltpu.VMEM((WROWS, CHUNK), jnp.int32),
            pltpu.VMEM((CHUNK, DEGW), jnp.float32),
            pltpu.VMEM((ZROWS, DEGW), jnp.float32),
            pltpu.VMEM_SHARED((NP, DEGW), jnp.float32),
            pltpu.SemaphoreType.DMA,
        ],
        interpret=interpret,
    )


def _sc_degree_body(dst2_hbm, out_hbm, dst2d, ones_v, zbuf, acc, ssem):
    c = lax.axis_index("c")
    s = lax.axis_index("s")
    w = s * NC + c

    _zero_vmem(zbuf, ZROWS, DEGW)
    one = jnp.full((16,), 1.0, jnp.float32)
    for r in range(CHUNK):
        for j in range(DEGW // 16):
            ones_v[r, pl.ds(j * 16, 16)] = one

    def zloop(i, _):
        pltpu.sync_copy(zbuf, acc.at[pl.ds(s * ROWS_PER_SUB + i * ZROWS, ZROWS)])
        return ()

    lax.fori_loop(0, ROWS_PER_SUB // ZROWS, zloop, ())
    pltpu.sync_copy(dst2_hbm.at[pl.ds(w * WROWS, WROWS)], dst2d)
    plsc.subcore_barrier()

    def fire(i, _):
        pltpu.async_copy(ones_v, acc.at[dst2d.at[i]], ssem, add=True)

        @pl.when(i >= SWIN)
        def _drain_old():
            pltpu.make_async_copy(ones_v, acc.at[dst2d.at[i - SWIN]], ssem).wait()

        return ()

    lax.fori_loop(0, WROWS, fire, ())

    def drain(i, _):
        pltpu.make_async_copy(ones_v, acc.at[dst2d.at[i]], ssem).wait()
        return ()

    lax.fori_loop(WROWS - SWIN, WROWS, drain, ())

    plsc.subcore_barrier()
    pltpu.sync_copy(
        acc.at[pl.ds(s * ROWS_PER_SUB, ROWS_PER_SUB)],
        out_hbm.at[c, pl.ds(s * ROWS_PER_SUB, ROWS_PER_SUB)],
    )


def _make_sc_aggregate(interpret=False):
    return pl.kernel(
        _sc_aggregate_body,
        out_type=jax.ShapeDtypeStruct((NC, NP, D), jnp.float32),
        mesh=_sc_mesh,
        scratch_types=[
            pltpu.VMEM((HROWS, CHUNK), jnp.int32),
            pltpu.VMEM((HROWS, CHUNK), jnp.int32),
            pltpu.VMEM((CHUNK, D), jnp.float32),
            pltpu.VMEM((CHUNK, D), jnp.float32),
            pltpu.VMEM_SHARED((NP, D), jnp.float32),
            pltpu.SemaphoreType.DMA,
            pltpu.SemaphoreType.DMA,
        ],
        interpret=interpret,
    )


def _sc_aggregate_body(
    hs_hbm, src2_hbm, dst2_hbm, out_hbm,
    src2d, dst2d, rows0, rows1, acc, gsem, ssem,
):
    c = lax.axis_index("c")
    s = lax.axis_index("s")
    w = s * NC + c

    _zero_vmem(rows0, CHUNK, D)

    def zloop(i, _):
        pltpu.sync_copy(rows0, acc.at[pl.ds(s * ROWS_PER_SUB + i * CHUNK, CHUNK)])
        return ()

    lax.fori_loop(0, ROWS_PER_SUB // CHUNK, zloop, ())

    def gather(ci, buf):
        pltpu.async_copy(hs_hbm.at[src2d.at[ci]], buf, gsem)

    def wait_gather(ci, buf):
        pltpu.make_async_copy(hs_hbm.at[src2d.at[ci]], buf, gsem).wait()

    def scatter(ci, buf):
        pltpu.async_copy(buf, acc.at[dst2d.at[ci]], ssem, add=True)

    def wait_scatter(ci, buf):
        pltpu.make_async_copy(buf, acc.at[dst2d.at[ci]], ssem).wait()

    def pair(c2, _):
        c0 = 2 * c2
        c1 = c0 + 1
        wait_gather(c0, rows0)

        @pl.when(c2 > 0)
        def _wait_prev():
            wait_scatter(c0 - 1, rows1)

        gather(c1, rows1)
        scatter(c0, rows0)
        wait_gather(c1, rows1)
        wait_scatter(c0, rows0)

        @pl.when(c2 < HROWS // 2 - 1)
        def _prefetch_next():
            gather(c1 + 1, rows0)

        scatter(c1, rows1)
        return ()

    for h in range(WROWS // HROWS):
        pltpu.sync_copy(src2_hbm.at[pl.ds(w * WROWS + h * HROWS, HROWS)], src2d)
        pltpu.sync_copy(dst2_hbm.at[pl.ds(w * WROWS + h * HROWS, HROWS)], dst2d)
        if h == 0:
            plsc.subcore_barrier()
        gather(0, rows0)
        lax.fori_loop(0, HROWS // 2, pair, ())
        wait_scatter(HROWS - 1, rows1)

    plsc.subcore_barrier()
    pltpu.sync_copy(
        acc.at[pl.ds(s * ROWS_PER_SUB, ROWS_PER_SUB)],
        out_hbm.at[c, pl.ds(s * ROWS_PER_SUB, ROWS_PER_SUB)],
    )


_sc_degree = _make_sc_degree()
_sc_aggregate = _make_sc_aggregate()


ROWBLK = 1024
GRID = NP // ROWBLK

_row_spec = pl.BlockSpec((ROWBLK, D), lambda i: (i, 0))
_p_spec = pl.BlockSpec((ROWBLK, DEGW), lambda i: (i, 0))
_dinv_spec = pl.BlockSpec((ROWBLK, DINVW), lambda i: (i, 0))
_w_spec = pl.BlockSpec((D, D), lambda i: (0, 0))
_b_spec = pl.BlockSpec((1, D), lambda i: (0, 0))


def _leaky(y):
    return jnp.where(y >= 0, y, 0.01 * y)


def _tc_first_body(x_ref, w_ref, p0_ref, p1_ref, hs_ref, dinv_ref):
    deg = p0_ref[:, 0:1] + p1_ref[:, 0:1] + 1.0
    dinv = lax.rsqrt(jnp.maximum(deg, 1.0))
    hs_ref[...] = dinv * jnp.dot(
        x_ref[...], w_ref[...], preferred_element_type=jnp.float32
    )
    dinv_ref[...] = jnp.broadcast_to(dinv, (ROWBLK, DINVW))


def _tc_first(x, w1, p0, p1):
    return pl.pallas_call(
        _tc_first_body,
        grid=(GRID,),
        in_specs=[_row_spec, _w_spec, _p_spec, _p_spec],
        out_specs=[_row_spec, _dinv_spec],
        out_shape=[
            jax.ShapeDtypeStruct((NP, D), jnp.float32),
            jax.ShapeDtypeStruct((NP, DINVW), jnp.float32),
        ],
    )(x, w1, p0, p1)


def _tc_mid_body(p0_ref, p1_ref, hs_ref, b_ref, dinv_ref, w_ref, out_ref):
    dinv = dinv_ref[:, 0:1]
    y = dinv * (p0_ref[...] + p1_ref[...] + hs_ref[...]) + b_ref[...]
    xn = _leaky(y)
    out_ref[...] = dinv * jnp.dot(
        xn, w_ref[...], preferred_element_type=jnp.float32
    )


def _tc_mid(p0, p1, hs, b, dinvb, w):
    return pl.pallas_call(
        _tc_mid_body,
        grid=(GRID,),
        in_specs=[_row_spec, _row_spec, _row_spec, _b_spec, _dinv_spec, _w_spec],
        out_specs=_row_spec,
        out_shape=jax.ShapeDtypeStruct((NP, D), jnp.float32),
    )(p0, p1, hs, b, dinvb, w)


def _tc_last_body(p0_ref, p1_ref, hs_ref, b_ref, dinv_ref, out_ref):
    dinv = dinv_ref[:, 0:1]
    y = dinv * (p0_ref[...] + p1_ref[...] + hs_ref[...]) + b_ref[...]
    out_ref[...] = _leaky(y)


def _tc_last(p0, p1, hs, b, dinvb):
    return pl.pallas_call(
        _tc_last_body,
        grid=(GRID,),
        in_specs=[_row_spec, _row_spec, _row_spec, _b_spec, _dinv_spec],
        out_specs=_row_spec,
        out_shape=jax.ShapeDtypeStruct((NP, D), jnp.float32),
    )(p0, p1, hs, b, dinvb)


def kernel(x, edge_index, W1, b1, W2, b2, W3, b3, W4, b4):
    src = edge_index[0].astype(jnp.int32)
    dst = edge_index[1].astype(jnp.int32)
    pad = N + (jnp.arange(EPAD - E, dtype=jnp.int32) % (NP - N))
    src2 = jnp.concatenate([src, pad]).reshape(EROWS, CHUNK)
    dst2 = jnp.concatenate([dst, pad]).reshape(EROWS, CHUNK)
    xp = jnp.pad(x, ((0, NP - N), (0, 0)))

    dpart = _sc_degree(dst2)
    hs, dinvb = _tc_first(xp, W1, dpart[0], dpart[1])

    for w, b in ((W2, b1), (W3, b2), (W4, b3)):
        p = _sc_aggregate(hs, src2, dst2)
        hs = _tc_mid(p[0], p[1], hs, b.reshape(1, D), dinvb, w)

    p = _sc_aggregate(hs, src2, dst2)
    out = _tc_last(p[0], p[1], hs, b4.reshape(1, D), dinvb)
    return out[:N]


# speedup vs baseline: 2.9099x; 1.0668x over previous
import jax
import jax.numpy as jnp
from jax import lax
from jax.experimental import pallas as pl
from jax.experimental.pallas import tpu as pltpu
from jax.experimental.pallas import tpu_sc as plsc

N = 10000
NP = 10240
D = 128
E = 320000

NC = 2
NS = 16
NW = NC * NS
CHUNK = 128
WROWS = 80
EROWS = NW * WROWS
EPAD = EROWS * CHUNK
HROWS = WROWS // 2
ROWS_PER_SUB = NP // NS
ZROWS = 32
SWIN = 8
DEGW = 128
DINVW = 16

_sc_mesh = plsc.VectorSubcoreMesh(
    core_axis_name="c", subcore_axis_name="s", num_cores=NC, num_subcores=NS
)


def _zero_vmem(buf, rows, width):
    zero = jnp.zeros((16,), jnp.float32)
    for r in range(rows):
        for j in range(width // 16):
            buf[r, pl.ds(j * 16, 16)] = zero


def _make_sc_degree(interpret=False):
    return pl.kernel(
        _sc_degree_body,
        out_type=jax.ShapeDtypeStruct((NC, NP, DEGW), jnp.float32),
        mesh=_sc_mesh,
        scratch_types=[
            pltpu.VMEM((WROWS, CHUNK), jnp.int32),
            pltpu.VMEM((CHUNK, DEGW), jnp.float32),
            pltpu.VMEM((ZROWS, DEGW), jnp.float32),
            pltpu.VMEM_SHARED((NP, DEGW), jnp.float32),
            pltpu.SemaphoreType.DMA,
        ],
        interpret=interpret,
    )


def _sc_degree_body(dst2_hbm, out_hbm, dst2d, ones_v, zbuf, acc, ssem):
    c = lax.axis_index("c")
    s = lax.axis_index("s")
    w = s * NC + c

    _zero_vmem(zbuf, ZROWS, DEGW)
    one = jnp.full((16,), 1.0, jnp.float32)
    for r in range(CHUNK):
        for j in range(DEGW // 16):
            ones_v[r, pl.ds(j * 16, 16)] = one

    def zloop(i, _):
        pltpu.sync_copy(zbuf, acc.at[pl.ds(s * ROWS_PER_SUB + i * ZROWS, ZROWS)])
        return ()

    lax.fori_loop(0, ROWS_PER_SUB // ZROWS, zloop, ())
    pltpu.sync_copy(dst2_hbm.at[pl.ds(w * WROWS, WROWS)], dst2d)
    plsc.subcore_barrier()

    def fire(i, _):
        pltpu.async_copy(ones_v, acc.at[dst2d.at[i]], ssem, add=True)

        @pl.when(i >= SWIN)
        def _drain_old():
            pltpu.make_async_copy(ones_v, acc.at[dst2d.at[i - SWIN]], ssem).wait()

        return ()

    lax.fori_loop(0, WROWS, fire, ())

    def drain(i, _):
        pltpu.make_async_copy(ones_v, acc.at[dst2d.at[i]], ssem).wait()
        return ()

    lax.fori_loop(WROWS - SWIN, WROWS, drain, ())

    plsc.subcore_barrier()
    pltpu.sync_copy(
        acc.at[pl.ds(s * ROWS_PER_SUB, ROWS_PER_SUB)],
        out_hbm.at[c, pl.ds(s * ROWS_PER_SUB, ROWS_PER_SUB)],
    )


def _make_sc_aggregate(interpret=False):
    return pl.kernel(
        _sc_aggregate_body,
        out_type=jax.ShapeDtypeStruct((NC, NP, D), jnp.float32),
        mesh=_sc_mesh,
        scratch_types=[
            pltpu.VMEM((HROWS, CHUNK), jnp.int32),
            pltpu.VMEM((HROWS, CHUNK), jnp.int32),
            pltpu.VMEM((CHUNK, D), jnp.float32),
            pltpu.VMEM((CHUNK, D), jnp.float32),
            pltpu.VMEM_SHARED((NP, D), jnp.float32),
            pltpu.SemaphoreType.DMA,
            pltpu.SemaphoreType.DMA,
        ],
        interpret=interpret,
    )


def _sc_aggregate_body(
    hs_hbm, src2_hbm, dst2_hbm, out_hbm,
    src2d, dst2d, rows0, rows1, acc, gsem, ssem,
):
    c = lax.axis_index("c")
    s = lax.axis_index("s")
    w = s * NC + c

    def gather(ci, buf):
        pltpu.async_copy(hs_hbm.at[src2d.at[ci]], buf, gsem)

    def wait_gather(ci, buf):
        pltpu.make_async_copy(hs_hbm.at[src2d.at[ci]], buf, gsem).wait()

    def scatter(ci, buf):
        pltpu.async_copy(buf, acc.at[dst2d.at[ci]], ssem, add=True)

    def wait_scatter(ci, buf):
        pltpu.make_async_copy(buf, acc.at[dst2d.at[ci]], ssem).wait()

    pltpu.sync_copy(src2_hbm.at[pl.ds(w * WROWS, HROWS)], src2d)
    pltpu.sync_copy(dst2_hbm.at[pl.ds(w * WROWS, HROWS)], dst2d)
    gather(0, rows0)

    _zero_vmem(rows1, CHUNK, D)

    def zloop(i, _):
        pltpu.sync_copy(rows1, acc.at[pl.ds(s * ROWS_PER_SUB + i * CHUNK, CHUNK)])
        return ()

    lax.fori_loop(0, ROWS_PER_SUB // CHUNK, zloop, ())

    def pair(c2, _):
        c0 = 2 * c2
        c1 = c0 + 1
        wait_gather(c0, rows0)
        scatter(c0, rows0)

        @pl.when(c2 > 0)
        def _wait_prev():
            wait_scatter(c0 - 1, rows1)

        gather(c1, rows1)
        wait_gather(c1, rows1)
        scatter(c1, rows1)
        wait_scatter(c0, rows0)

        @pl.when(c2 < HROWS // 2 - 1)
        def _prefetch_next():
            gather(c1 + 1, rows0)

        return ()

    for h in range(WROWS // HROWS):
        if h == 0:
            plsc.subcore_barrier()
        else:
            pltpu.sync_copy(
                src2_hbm.at[pl.ds(w * WROWS + h * HROWS, HROWS)], src2d
            )
            pltpu.sync_copy(
                dst2_hbm.at[pl.ds(w * WROWS + h * HROWS, HROWS)], dst2d
            )
            gather(0, rows0)
        lax.fori_loop(0, HROWS // 2, pair, ())
        wait_scatter(HROWS - 1, rows1)

    plsc.subcore_barrier()
    pltpu.sync_copy(
        acc.at[pl.ds(s * ROWS_PER_SUB, ROWS_PER_SUB)],
        out_hbm.at[c, pl.ds(s * ROWS_PER_SUB, ROWS_PER_SUB)],
    )


_sc_degree = _make_sc_degree()
_sc_aggregate = _make_sc_aggregate()


ROWBLK = 1000
GRID = N // ROWBLK

_row_spec = pl.BlockSpec((ROWBLK, D), lambda i: (i, 0))
_p0_spec = pl.BlockSpec((1, ROWBLK, DEGW), lambda i: (0, i, 0))
_p1_spec = pl.BlockSpec((1, ROWBLK, DEGW), lambda i: (1, i, 0))
_a0_spec = pl.BlockSpec((1, ROWBLK, D), lambda i: (0, i, 0))
_a1_spec = pl.BlockSpec((1, ROWBLK, D), lambda i: (1, i, 0))
_dinv_spec = pl.BlockSpec((ROWBLK, DINVW), lambda i: (i, 0))
_w_spec = pl.BlockSpec((D, D), lambda i: (0, 0))
_b_spec = pl.BlockSpec((1, D), lambda i: (0, 0))


def _leaky(y):
    return jnp.where(y >= 0, y, 0.01 * y)


def _tc_first_body(x_ref, w_ref, p0_ref, p1_ref, hs_ref, dinv_ref):
    deg = p0_ref[0, :, 0:1] + p1_ref[0, :, 0:1] + 1.0
    dinv = lax.rsqrt(jnp.maximum(deg, 1.0))
    hs_ref[...] = dinv * jnp.dot(
        x_ref[...], w_ref[...], preferred_element_type=jnp.float32
    )
    dinv_ref[...] = jnp.broadcast_to(dinv, (ROWBLK, DINVW))


def _tc_first(x, w1, dpart):
    return pl.pallas_call(
        _tc_first_body,
        grid=(GRID,),
        in_specs=[_row_spec, _w_spec, _p0_spec, _p1_spec],
        out_specs=[_row_spec, _dinv_spec],
        out_shape=[
            jax.ShapeDtypeStruct((NP, D), jnp.float32),
            jax.ShapeDtypeStruct((NP, DINVW), jnp.float32),
        ],
    )(x, w1, dpart, dpart)


def _tc_mid_body(p0_ref, p1_ref, hs_ref, b_ref, dinv_ref, w_ref, out_ref):
    dinv = dinv_ref[:, 0:1]
    y = dinv * (p0_ref[0] + p1_ref[0] + hs_ref[...]) + b_ref[...]
    xn = _leaky(y)
    out_ref[...] = dinv * jnp.dot(
        xn, w_ref[...], preferred_element_type=jnp.float32
    )


def _tc_mid(p, hs, b, dinvb, w):
    return pl.pallas_call(
        _tc_mid_body,
        grid=(GRID,),
        in_specs=[_a0_spec, _a1_spec, _row_spec, _b_spec, _dinv_spec, _w_spec],
        out_specs=_row_spec,
        out_shape=jax.ShapeDtypeStruct((NP, D), jnp.float32),
    )(p, p, hs, b, dinvb, w)


def _tc_last_body(p0_ref, p1_ref, hs_ref, b_ref, dinv_ref, out_ref):
    dinv = dinv_ref[:, 0:1]
    y = dinv * (p0_ref[0] + p1_ref[0] + hs_ref[...]) + b_ref[...]
    out_ref[...] = _leaky(y)


def _tc_last(p, hs, b, dinvb):
    return pl.pallas_call(
        _tc_last_body,
        grid=(GRID,),
        in_specs=[_a0_spec, _a1_spec, _row_spec, _b_spec, _dinv_spec],
        out_specs=_row_spec,
        out_shape=jax.ShapeDtypeStruct((N, D), jnp.float32),
    )(p, p, hs, b, dinvb)


def kernel(x, edge_index, W1, b1, W2, b2, W3, b3, W4, b4):
    src = edge_index[0].astype(jnp.int32)
    dst = edge_index[1].astype(jnp.int32)
    pad = N + (jnp.arange(EPAD - E, dtype=jnp.int32) % (NP - N))
    src2 = jnp.concatenate([src, pad]).reshape(EROWS, CHUNK)
    dst2 = jnp.concatenate([dst, pad]).reshape(EROWS, CHUNK)

    dpart = _sc_degree(dst2)
    hs, dinvb = _tc_first(x, W1, dpart)

    for w, b in ((W2, b1), (W3, b2), (W4, b3)):
        p = _sc_aggregate(hs, src2, dst2)
        hs = _tc_mid(p, hs, b.reshape(1, D), dinvb, w)

    p = _sc_aggregate(hs, src2, dst2)
    return _tc_last(p, hs, b4.reshape(1, D), dinvb)


# speedup vs baseline: 3.2112x; 1.1036x over previous
import jax
import jax.numpy as jnp
from jax import lax
from jax.experimental import pallas as pl
from jax.experimental.pallas import tpu as pltpu
from jax.experimental.pallas import tpu_sc as plsc

N = 10000
NP = 10240
D = 128
E = 320000

NC = 2
NS = 16
NW = NC * NS
CHUNK = 128
WROWS = 80
EROWS = NW * WROWS
EPAD = EROWS * CHUNK
HROWS = WROWS // 2
ROWS_PER_SUB = NP // NS
ZROWS = 32
SWIN = 8
DEGW = 128
DINVW = 16

_sc_mesh = plsc.VectorSubcoreMesh(
    core_axis_name="c", subcore_axis_name="s", num_cores=NC, num_subcores=NS
)


def _zero_vmem(buf, rows, width):
    zero = jnp.zeros((16,), jnp.float32)
    for r in range(rows):
        for j in range(width // 16):
            buf[r, pl.ds(j * 16, 16)] = zero


PACK = NP // 128
PACK_CHUNK = 8
PACK_SUBS = PACK // PACK_CHUNK


def _make_sc_degree(interpret=False):
    return pl.kernel(
        _sc_degree_body,
        out_type=jax.ShapeDtypeStruct((NC, PACK, 128), jnp.float32),
        mesh=_sc_mesh,
        scratch_types=[
            pltpu.VMEM((WROWS, CHUNK), jnp.int32),
            pltpu.VMEM((CHUNK,), jnp.float32),
            pltpu.VMEM((PACK_CHUNK * 128,), jnp.float32),
            pltpu.VMEM((PACK_CHUNK, 128), jnp.float32),
            pltpu.VMEM_SHARED((NP,), jnp.float32),
            pltpu.SemaphoreType.DMA,
        ],
        interpret=interpret,
    )


def _sc_degree_body(dst2_hbm, out_hbm, dst2d, ones1, buf1, buf2, acc1, ssem):
    c = lax.axis_index("c")
    s = lax.axis_index("s")
    w = s * NC + c

    pltpu.sync_copy(dst2_hbm.at[pl.ds(w * WROWS, WROWS)], dst2d)
    one = jnp.full((16,), 1.0, jnp.float32)
    zero = jnp.zeros((16,), jnp.float32)
    for g in range(CHUNK // 16):
        ones1[pl.ds(g * 16, 16)] = one
    for k in range(PACK_CHUNK * 8):
        buf1[pl.ds(k * 16, 16)] = zero
    pltpu.sync_copy(
        buf1.at[pl.ds(0, ROWS_PER_SUB)],
        acc1.at[pl.ds(s * ROWS_PER_SUB, ROWS_PER_SUB)],
    )
    plsc.subcore_barrier()

    def fire(i, _):
        pltpu.async_copy(ones1, acc1.at[dst2d.at[i]], ssem, add=True)

        @pl.when(i >= SWIN)
        def _drain_old():
            pltpu.make_async_copy(ones1, acc1.at[dst2d.at[i - SWIN]], ssem).wait()

        return ()

    lax.fori_loop(0, WROWS, fire, ())

    def drain(i, _):
        pltpu.make_async_copy(ones1, acc1.at[dst2d.at[i]], ssem).wait()
        return ()

    lax.fori_loop(WROWS - SWIN, WROWS, drain, ())

    plsc.subcore_barrier()

    @pl.when(s < PACK_SUBS)
    def _copyout():
        n0 = s * PACK_CHUNK * 128
        pltpu.sync_copy(acc1.at[pl.ds(n0, PACK_CHUNK * 128)], buf1)
        for k in range(PACK_CHUNK * 8):
            r, g = divmod(k, 8)
            buf2[r, pl.ds(g * 16, 16)] = buf1[pl.ds(k * 16, 16)]
        pltpu.sync_copy(buf2, out_hbm.at[c, pl.ds(s * PACK_CHUNK, PACK_CHUNK)])


def _make_sc_aggregate(interpret=False):
    return pl.kernel(
        _sc_aggregate_body,
        out_type=jax.ShapeDtypeStruct((NC, NP, D), jnp.float32),
        mesh=_sc_mesh,
        scratch_types=[
            pltpu.VMEM((HROWS, CHUNK), jnp.int32),
            pltpu.VMEM((HROWS, CHUNK), jnp.int32),
            pltpu.VMEM((CHUNK, D), jnp.float32),
            pltpu.VMEM((CHUNK, D), jnp.float32),
            pltpu.VMEM_SHARED((NP, D), jnp.float32),
            pltpu.SemaphoreType.DMA,
            pltpu.SemaphoreType.DMA,
        ],
        interpret=interpret,
    )


def _sc_aggregate_body(
    hs_hbm, src2_hbm, dst2_hbm, out_hbm,
    src2d, dst2d, rows0, rows1, acc, gsem, ssem,
):
    c = lax.axis_index("c")
    s = lax.axis_index("s")
    w = s * NC + c

    def gather(ci, buf):
        pltpu.async_copy(hs_hbm.at[src2d.at[ci]], buf, gsem)

    def wait_gather(ci, buf):
        pltpu.make_async_copy(hs_hbm.at[src2d.at[ci]], buf, gsem).wait()

    def scatter(ci, buf):
        pltpu.async_copy(buf, acc.at[dst2d.at[ci]], ssem, add=True)

    def wait_scatter(ci, buf):
        pltpu.make_async_copy(buf, acc.at[dst2d.at[ci]], ssem).wait()

    pltpu.sync_copy(src2_hbm.at[pl.ds(w * WROWS, HROWS)], src2d)
    pltpu.sync_copy(dst2_hbm.at[pl.ds(w * WROWS, HROWS)], dst2d)
    gather(0, rows0)

    _zero_vmem(rows1, CHUNK, D)

    def zloop(i, _):
        pltpu.sync_copy(rows1, acc.at[pl.ds(s * ROWS_PER_SUB + i * CHUNK, CHUNK)])
        return ()

    lax.fori_loop(0, ROWS_PER_SUB // CHUNK, zloop, ())

    def pair(c2, _):
        c0 = 2 * c2
        c1 = c0 + 1
        wait_gather(c0, rows0)
        scatter(c0, rows0)

        @pl.when(c2 > 0)
        def _wait_prev():
            wait_scatter(c0 - 1, rows1)

        gather(c1, rows1)
        wait_gather(c1, rows1)
        scatter(c1, rows1)
        wait_scatter(c0, rows0)

        @pl.when(c2 < HROWS // 2 - 1)
        def _prefetch_next():
            gather(c1 + 1, rows0)

        return ()

    for h in range(WROWS // HROWS):
        if h == 0:
            plsc.subcore_barrier()
        else:
            pltpu.sync_copy(
                src2_hbm.at[pl.ds(w * WROWS + h * HROWS, HROWS)], src2d
            )
            pltpu.sync_copy(
                dst2_hbm.at[pl.ds(w * WROWS + h * HROWS, HROWS)], dst2d
            )
            gather(0, rows0)
        lax.fori_loop(0, HROWS // 2, pair, ())
        wait_scatter(HROWS - 1, rows1)

    plsc.subcore_barrier()
    pltpu.sync_copy(
        acc.at[pl.ds(s * ROWS_PER_SUB, ROWS_PER_SUB)],
        out_hbm.at[c, pl.ds(s * ROWS_PER_SUB, ROWS_PER_SUB)],
    )


_sc_degree = _make_sc_degree()
_sc_aggregate = _make_sc_aggregate()


ROWBLK = 1000
GRID = N // ROWBLK

_row_spec = pl.BlockSpec((ROWBLK, D), lambda i: (i, 0))
_a0_spec = pl.BlockSpec((1, ROWBLK, D), lambda i: (0, i, 0))
_a1_spec = pl.BlockSpec((1, ROWBLK, D), lambda i: (1, i, 0))
_dinv_spec = pl.BlockSpec((ROWBLK, DINVW), lambda i: (i, 0))
_w_spec = pl.BlockSpec((D, D), lambda i: (0, 0))
_b_spec = pl.BlockSpec((1, D), lambda i: (0, 0))

FBLK = 1024
FGRID = NP // FBLK
_frow_spec = pl.BlockSpec((FBLK, D), lambda i: (i, 0))
_fdinv_spec = pl.BlockSpec((FBLK, DINVW), lambda i: (i, 0))
_d0_spec = pl.BlockSpec((1, 8, 128), lambda i: (0, i, 0))
_d1_spec = pl.BlockSpec((1, 8, 128), lambda i: (1, i, 0))


def _leaky(y):
    return jnp.where(y >= 0, y, 0.01 * y)


def _tc_first_body(x_ref, w_ref, p0_ref, p1_ref, hs_ref, dinv_ref):
    deg8 = p0_ref[0] + p1_ref[0] + 1.0
    dinv8 = lax.rsqrt(jnp.maximum(deg8, 1.0))
    dinv = jnp.concatenate(
        [jnp.transpose(dinv8[q : q + 1, :]) for q in range(8)], axis=0
    )
    hs_ref[...] = dinv * jnp.dot(
        x_ref[...], w_ref[...], preferred_element_type=jnp.float32
    )
    dinv_ref[...] = jnp.broadcast_to(dinv, (FBLK, DINVW))


def _tc_first(x, w1, dpart):
    return pl.pallas_call(
        _tc_first_body,
        grid=(FGRID,),
        in_specs=[_frow_spec, _w_spec, _d0_spec, _d1_spec],
        out_specs=[_frow_spec, _fdinv_spec],
        out_shape=[
            jax.ShapeDtypeStruct((NP, D), jnp.float32),
            jax.ShapeDtypeStruct((NP, DINVW), jnp.float32),
        ],
    )(x, w1, dpart, dpart)


def _tc_mid_body(p0_ref, p1_ref, hs_ref, b_ref, dinv_ref, w_ref, out_ref):
    dinv = dinv_ref[:, 0:1]
    y = dinv * (p0_ref[0] + p1_ref[0] + hs_ref[...]) + b_ref[...]
    xn = _leaky(y)
    out_ref[...] = dinv * jnp.dot(
        xn, w_ref[...], preferred_element_type=jnp.float32
    )


def _tc_mid(p, hs, b, dinvb, w):
    return pl.pallas_call(
        _tc_mid_body,
        grid=(GRID,),
        in_specs=[_a0_spec, _a1_spec, _row_spec, _b_spec, _dinv_spec, _w_spec],
        out_specs=_row_spec,
        out_shape=jax.ShapeDtypeStruct((NP, D), jnp.float32),
    )(p, p, hs, b, dinvb, w)


def _tc_last_body(p0_ref, p1_ref, hs_ref, b_ref, dinv_ref, out_ref):
    dinv = dinv_ref[:, 0:1]
    y = dinv * (p0_ref[0] + p1_ref[0] + hs_ref[...]) + b_ref[...]
    out_ref[...] = _leaky(y)


def _tc_last(p, hs, b, dinvb):
    return pl.pallas_call(
        _tc_last_body,
        grid=(GRID,),
        in_specs=[_a0_spec, _a1_spec, _row_spec, _b_spec, _dinv_spec],
        out_specs=_row_spec,
        out_shape=jax.ShapeDtypeStruct((N, D), jnp.float32),
    )(p, p, hs, b, dinvb)


def kernel(x, edge_index, W1, b1, W2, b2, W3, b3, W4, b4):
    src = edge_index[0].astype(jnp.int32)
    dst = edge_index[1].astype(jnp.int32)
    pad = N + (jnp.arange(EPAD - E, dtype=jnp.int32) % (NP - N))
    src2 = jnp.concatenate([src, pad]).reshape(EROWS, CHUNK)
    dst2 = jnp.concatenate([dst, pad]).reshape(EROWS, CHUNK)
    xp = jnp.pad(x, ((0, NP - N), (0, 0)))

    dpart = _sc_degree(dst2)
    hs, dinvb = _tc_first(xp, W1, dpart)

    for w, b in ((W2, b1), (W3, b2), (W4, b3)):
        p = _sc_aggregate(hs, src2, dst2)
        hs = _tc_mid(p, hs, b.reshape(1, D), dinvb, w)

    p = _sc_aggregate(hs, src2, dst2)
    return _tc_last(p, hs, b4.reshape(1, D), dinvb)


# speedup vs baseline: 3.2367x; 1.0079x over previous
import jax
import jax.numpy as jnp
from jax import lax
from jax.experimental import pallas as pl
from jax.experimental.pallas import tpu as pltpu
from jax.experimental.pallas import tpu_sc as plsc

N = 10000
NP = 10240
D = 128
E = 320000

NC = 2
NS = 16
NW = NC * NS
CHUNK = 128
WROWS = 80
EROWS = NW * WROWS
EPAD = EROWS * CHUNK
HROWS = WROWS // 2
ROWS_PER_SUB = NP // NS
ZROWS = 32
SWIN = 8
DEGW = 128
DINVW = 16

_sc_mesh = plsc.VectorSubcoreMesh(
    core_axis_name="c", subcore_axis_name="s", num_cores=NC, num_subcores=NS
)


def _zero_vmem(buf, rows, width):
    zero = jnp.zeros((16,), jnp.float32)
    for r in range(rows):
        for j in range(width // 16):
            buf[r, pl.ds(j * 16, 16)] = zero


PACK = NP // 128
PACK_CHUNK = 8
PACK_SUBS = PACK // PACK_CHUNK


def _make_sc_degree(interpret=False):
    return pl.kernel(
        _sc_degree_body,
        out_type=jax.ShapeDtypeStruct((NC, PACK, 128), jnp.float32),
        mesh=_sc_mesh,
        scratch_types=[
            pltpu.VMEM((WROWS, CHUNK), jnp.int32),
            pltpu.VMEM((CHUNK,), jnp.float32),
            pltpu.VMEM((PACK_CHUNK * 128,), jnp.float32),
            pltpu.VMEM((PACK_CHUNK, 128), jnp.float32),
            pltpu.VMEM_SHARED((NP,), jnp.float32),
            pltpu.SemaphoreType.DMA,
        ],
        interpret=interpret,
    )


def _sc_degree_body(dst2_hbm, out_hbm, dst2d, ones1, buf1, buf2, acc1, ssem):
    c = lax.axis_index("c")
    s = lax.axis_index("s")
    w = s * NC + c

    pltpu.sync_copy(dst2_hbm.at[pl.ds(w * WROWS, WROWS)], dst2d)
    one = jnp.full((16,), 1.0, jnp.float32)
    zero = jnp.zeros((16,), jnp.float32)
    for g in range(CHUNK // 16):
        ones1[pl.ds(g * 16, 16)] = one
    for k in range(PACK_CHUNK * 8):
        buf1[pl.ds(k * 16, 16)] = zero
    pltpu.sync_copy(
        buf1.at[pl.ds(0, ROWS_PER_SUB)],
        acc1.at[pl.ds(s * ROWS_PER_SUB, ROWS_PER_SUB)],
    )
    plsc.subcore_barrier()

    def fire(i, _):
        pltpu.async_copy(ones1, acc1.at[dst2d.at[i]], ssem, add=True)

        @pl.when(i >= SWIN)
        def _drain_old():
            pltpu.make_async_copy(ones1, acc1.at[dst2d.at[i - SWIN]], ssem).wait()

        return ()

    lax.fori_loop(0, WROWS, fire, ())

    def drain(i, _):
        pltpu.make_async_copy(ones1, acc1.at[dst2d.at[i]], ssem).wait()
        return ()

    lax.fori_loop(WROWS - SWIN, WROWS, drain, ())

    plsc.subcore_barrier()

    @pl.when(s < PACK_SUBS)
    def _copyout():
        n0 = s * PACK_CHUNK * 128
        pltpu.sync_copy(acc1.at[pl.ds(n0, PACK_CHUNK * 128)], buf1)
        for k in range(PACK_CHUNK * 8):
            r, g = divmod(k, 8)
            buf2[r, pl.ds(g * 16, 16)] = buf1[pl.ds(k * 16, 16)]
        pltpu.sync_copy(buf2, out_hbm.at[c, pl.ds(s * PACK_CHUNK, PACK_CHUNK)])


GROWS = 16
NGRP = WROWS // GROWS


def _make_sc_aggregate(interpret=False):
    return pl.kernel(
        _sc_aggregate_body,
        out_type=jax.ShapeDtypeStruct((NC, NP, D), jnp.float32),
        mesh=_sc_mesh,
        scratch_types=[
            pltpu.VMEM((GROWS, CHUNK), jnp.int32),
            pltpu.VMEM((GROWS, CHUNK), jnp.int32),
            pltpu.VMEM((GROWS, CHUNK), jnp.int32),
            pltpu.VMEM((GROWS, CHUNK), jnp.int32),
            pltpu.VMEM((CHUNK, D), jnp.float32),
            pltpu.VMEM((CHUNK, D), jnp.float32),
            pltpu.VMEM_SHARED((NP, D), jnp.float32),
            pltpu.SemaphoreType.DMA,
            pltpu.SemaphoreType.DMA,
            pltpu.SemaphoreType.DMA,
        ],
        interpret=interpret,
    )


def _sc_aggregate_body(
    hs_hbm, src2_hbm, dst2_hbm, out_hbm,
    srcA, dstA, srcB, dstB, rows0, rows1, acc, gsem, ssem, isem,
):
    c = lax.axis_index("c")
    s = lax.axis_index("s")
    w = s * NC + c
    idx = ((srcA, dstA), (srcB, dstB))

    def gather(sref, ci, buf):
        pltpu.async_copy(hs_hbm.at[sref.at[ci]], buf, gsem)

    def scatter(dref, ci, buf):
        pltpu.async_copy(buf, acc.at[dref.at[ci]], ssem, add=True)

    def wait_gather(buf):
        pltpu.make_async_copy(hs_hbm.at[srcA.at[0]], buf, gsem).wait()

    def wait_scatter(buf):
        pltpu.make_async_copy(buf, acc.at[dstA.at[0]], ssem).wait()

    def load_idx_group(g, bufs, sync):
        sref, dref = bufs
        if sync:
            pltpu.sync_copy(src2_hbm.at[pl.ds(w * WROWS + g * GROWS, GROWS)], sref)
            pltpu.sync_copy(dst2_hbm.at[pl.ds(w * WROWS + g * GROWS, GROWS)], dref)
        else:
            pltpu.async_copy(
                src2_hbm.at[pl.ds(w * WROWS + g * GROWS, GROWS)], sref, isem
            )
            pltpu.async_copy(
                dst2_hbm.at[pl.ds(w * WROWS + g * GROWS, GROWS)], dref, isem
            )

    def wait_idx_group(g, bufs):
        sref, dref = bufs
        pltpu.make_async_copy(
            src2_hbm.at[pl.ds(w * WROWS + g * GROWS, GROWS)], sref, isem
        ).wait()
        pltpu.make_async_copy(
            dst2_hbm.at[pl.ds(w * WROWS + g * GROWS, GROWS)], dref, isem
        ).wait()

    load_idx_group(0, idx[0], sync=True)
    gather(srcA, 0, rows0)

    _zero_vmem(rows1, CHUNK, D)

    def zloop(i, _):
        pltpu.sync_copy(rows1, acc.at[pl.ds(s * ROWS_PER_SUB + i * CHUNK, CHUNK)])
        return ()

    lax.fori_loop(0, ROWS_PER_SUB // CHUNK, zloop, ())

    def make_pair(bufs, first_group):
        sref, dref = bufs

        def pair(c2, _):
            c0 = 2 * c2
            c1 = c0 + 1
            wait_gather(rows0)
            scatter(dref, c0, rows0)

            if first_group:
                @pl.when(c2 > 0)
                def _wait_prev():
                    wait_scatter(rows1)
            else:
                wait_scatter(rows1)

            gather(sref, c1, rows1)
            wait_gather(rows1)
            scatter(dref, c1, rows1)
            wait_scatter(rows0)

            @pl.when(c2 < GROWS // 2 - 1)
            def _prefetch_next():
                gather(sref, c1 + 1, rows0)

            return ()

        return pair

    for g in range(NGRP):
        cur = idx[g % 2]
        nxt = idx[(g + 1) % 2]
        if g == 0:
            plsc.subcore_barrier()
        if g + 1 < NGRP:
            load_idx_group(g + 1, nxt, sync=False)
        lax.fori_loop(0, GROWS // 2, make_pair(cur, g == 0), ())
        if g + 1 < NGRP:
            wait_idx_group(g + 1, nxt)
            gather(nxt[0], 0, rows0)

    wait_scatter(rows1)

    plsc.subcore_barrier()
    pltpu.sync_copy(
        acc.at[pl.ds(s * ROWS_PER_SUB, ROWS_PER_SUB)],
        out_hbm.at[c, pl.ds(s * ROWS_PER_SUB, ROWS_PER_SUB)],
    )


_sc_degree = _make_sc_degree()
_sc_aggregate = _make_sc_aggregate()


ROWBLK = 1000
GRID = N // ROWBLK

_row_spec = pl.BlockSpec((ROWBLK, D), lambda i: (i, 0))
_a0_spec = pl.BlockSpec((1, ROWBLK, D), lambda i: (0, i, 0))
_a1_spec = pl.BlockSpec((1, ROWBLK, D), lambda i: (1, i, 0))
_dinv_spec = pl.BlockSpec((ROWBLK, DINVW), lambda i: (i, 0))
_w_spec = pl.BlockSpec((D, D), lambda i: (0, 0))
_b_spec = pl.BlockSpec((1, D), lambda i: (0, 0))

FBLK = 1024
FGRID = NP // FBLK
_frow_spec = pl.BlockSpec((FBLK, D), lambda i: (i, 0))
_fdinv_spec = pl.BlockSpec((FBLK, DINVW), lambda i: (i, 0))
_d0_spec = pl.BlockSpec((1, 8, 128), lambda i: (0, i, 0))
_d1_spec = pl.BlockSpec((1, 8, 128), lambda i: (1, i, 0))


def _leaky(y):
    return jnp.where(y >= 0, y, 0.01 * y)


def _tc_first_body(x_ref, w_ref, p0_ref, p1_ref, hs_ref, dinv_ref):
    deg8 = p0_ref[0] + p1_ref[0] + 1.0
    dinv8 = lax.rsqrt(jnp.maximum(deg8, 1.0))
    dinv = jnp.concatenate(
        [jnp.transpose(dinv8[q : q + 1, :]) for q in range(8)], axis=0
    )
    hs_ref[...] = dinv * jnp.dot(
        x_ref[...], w_ref[...], preferred_element_type=jnp.float32
    )
    dinv_ref[...] = jnp.broadcast_to(dinv, (FBLK, DINVW))


def _tc_first(x, w1, dpart):
    return pl.pallas_call(
        _tc_first_body,
        grid=(FGRID,),
        in_specs=[_frow_spec, _w_spec, _d0_spec, _d1_spec],
        out_specs=[_frow_spec, _fdinv_spec],
        out_shape=[
            jax.ShapeDtypeStruct((NP, D), jnp.float32),
            jax.ShapeDtypeStruct((NP, DINVW), jnp.float32),
        ],
    )(x, w1, dpart, dpart)


def _tc_mid_body(p0_ref, p1_ref, hs_ref, b_ref, dinv_ref, w_ref, out_ref):
    dinv = dinv_ref[:, 0:1]
    y = dinv * (p0_ref[0] + p1_ref[0] + hs_ref[...]) + b_ref[...]
    xn = _leaky(y)
    out_ref[...] = dinv * jnp.dot(
        xn, w_ref[...], preferred_element_type=jnp.float32
    )


def _tc_mid(p, hs, b, dinvb, w):
    return pl.pallas_call(
        _tc_mid_body,
        grid=(GRID,),
        in_specs=[_a0_spec, _a1_spec, _row_spec, _b_spec, _dinv_spec, _w_spec],
        out_specs=_row_spec,
        out_shape=jax.ShapeDtypeStruct((NP, D), jnp.float32),
    )(p, p, hs, b, dinvb, w)


def _tc_last_body(p0_ref, p1_ref, hs_ref, b_ref, dinv_ref, out_ref):
    dinv = dinv_ref[:, 0:1]
    y = dinv * (p0_ref[0] + p1_ref[0] + hs_ref[...]) + b_ref[...]
    out_ref[...] = _leaky(y)


def _tc_last(p, hs, b, dinvb):
    return pl.pallas_call(
        _tc_last_body,
        grid=(GRID,),
        in_specs=[_a0_spec, _a1_spec, _row_spec, _b_spec, _dinv_spec],
        out_specs=_row_spec,
        out_shape=jax.ShapeDtypeStruct((N, D), jnp.float32),
    )(p, p, hs, b, dinvb)


def kernel(x, edge_index, W1, b1, W2, b2, W3, b3, W4, b4):
    src = edge_index[0].astype(jnp.int32)
    dst = edge_index[1].astype(jnp.int32)
    pad = N + (jnp.arange(EPAD - E, dtype=jnp.int32) % (NP - N))
    src2 = jnp.concatenate([src, pad]).reshape(EROWS, CHUNK)
    dst2 = jnp.concatenate([dst, pad]).reshape(EROWS, CHUNK)
    xp = jnp.pad(x, ((0, NP - N), (0, 0)))

    dpart = _sc_degree(dst2)
    hs, dinvb = _tc_first(xp, W1, dpart)

    for w, b in ((W2, b1), (W3, b2), (W4, b3)):
        p = _sc_aggregate(hs, src2, dst2)
        hs = _tc_mid(p, hs, b.reshape(1, D), dinvb, w)

    p = _sc_aggregate(hs, src2, dst2)
    return _tc_last(p, hs, b4.reshape(1, D), dinvb)
